# baseline (speedup 1.0000x reference)
import dataclasses

import jax
import jax.numpy as jnp
from jax.experimental import pallas as pl
from jax.experimental.pallas import tpu as pltpu
from jax.experimental.pallas import tpu_sc as plsc

_ROWS = 128
_COLS = 32768
_TOTAL = _ROWS * _COLS
_KK = 64 * _ROWS
_CH = 8
_NCH = _ROWS // _CH
_SUB = 4096
_NSUB = _COLS // _SUB

_i32 = jnp.int32
_f32 = jnp.float32


def _lt(k, t):
    return jax.lax.shift_right_logical(k - t, 31)


_NTILES = 32
_BINS = 32768


def _sc_hist(x):
    mesh = plsc.VectorSubcoreMesh(core_axis_name="c", subcore_axis_name="s")

    cp = pltpu.CompilerParams()
    if "needs_layout_passes" in pltpu.CompilerParams.__dataclass_fields__:
        cp = dataclasses.replace(cp, needs_layout_passes=False)

    @pl.kernel(
        out_type=jax.ShapeDtypeStruct((_NTILES, _BINS), jnp.int32),
        mesh=mesh,
        scratch_types=[pltpu.VMEM((_BINS,), jnp.int32)],
        compiler_params=cp,
    )
    def hist_kernel(x_hbm, o_hbm, hist_ref):
        tile = jax.lax.axis_index("c") * 16 + jax.lax.axis_index("s")
        zeros16 = jnp.zeros((16,), jnp.int32)
        ones16 = jnp.ones((16,), jnp.int32)

        @pl.loop(0, _BINS // 16)
        def _(i):
            hist_ref[pl.ds(i * 16, 16)] = zeros16

        def body(in_vmem):
            @pl.loop(0, _COLS // 16)
            def _(i):
                v = in_vmem[0, pl.ds(i * 16, 16)]
                k = jnp.maximum(plsc.bitcast(v, jnp.int32), 0)
                b = jax.lax.shift_right_logical(k, 16)
                plsc.addupdate_scatter(hist_ref, [b], ones16)

        pltpu.emit_pipeline(
            body,
            grid=(_ROWS,),
            in_specs=[pl.BlockSpec((1, _COLS), lambda i: (i, 0))],
            core_axis_name=("c", "s"),
            dimension_semantics=(pltpu.PARALLEL,),
        )(x_hbm)

        pltpu.sync_copy(hist_ref, o_hbm.at[tile])

    return hist_kernel(x)


def _body(x_ref, h_ref, o_ref, hb_ref):
    for c in range(_NCH):
        xb = x_ref[c * _CH:(c + 1) * _CH, :]
        keys = jnp.maximum(jax.lax.bitcast_convert_type(xb, _i32), 0)
        o_ref[c * _CH:(c + 1) * _CH, :] = jax.lax.bitcast_convert_type(
            keys, _f32
        )

    def kvreg(c, s):
        return jax.lax.bitcast_convert_type(
            o_ref[c * _CH:(c + 1) * _CH, s * 128:(s + 1) * 128], _i32
        )

    def kchunk(c):
        return jax.lax.bitcast_convert_type(
            o_ref[c * _CH:(c + 1) * _CH, :], _i32
        )

    _NV = _COLS // 128

    def count_lt(t):
        accs = [jnp.zeros((_CH, 128), _i32) for _ in range(8)]
        i = 0
        for c in range(_NCH):
            for s in range(_NV):
                accs[i & 7] = accs[i & 7] + _lt(kvreg(c, s), t)
                i += 1
        a = accs
        while len(a) > 1:
            a = [a[j] + a[j + 1] for j in range(0, len(a), 2)]
        return jnp.sum(a[0])

    kk = jnp.int32(_KK)
    ge_kk = jnp.int32(_TOTAL - _KK)

    hb_ref[...] = (
        (h_ref[0:8, :] + h_ref[8:16, :])
        + (h_ref[16:24, :] + h_ref[24:32, :])
    )

    lane_iota = jax.lax.broadcasted_iota(_i32, (_CH, 128), 1)
    _NBV = _BINS // 128

    def hist_count_lt(b):
        accs = [jnp.zeros((_CH, 128), _i32) for _ in range(4)]
        for s in range(_NBV):
            m = _lt(lane_iota, b - s * 128)
            accs[s & 3] = accs[s & 3] + m * hb_ref[:, s * 128:(s + 1) * 128]
        return jnp.sum((accs[0] + accs[1]) + (accs[2] + accs[3]))

    def bin_round(i, cur):
        cand = cur + (jnp.int32(1) << (jnp.int32(14) - i))
        return jnp.where(hist_count_lt(cand) <= ge_kk, cand, cur)

    bstar = jax.lax.fori_loop(0, 15, bin_round, jnp.int32(0))

    def key_round(i, cur):
        cand = cur + (jnp.int32(1) << (jnp.int32(30) - i))
        return jnp.where(count_lt(cand) <= ge_kk, cand, cur)

    kstar = jax.lax.fori_loop(
        15, 31, key_round, jax.lax.shift_left(bstar, 16)
    )

    gaccs = [jnp.zeros((_CH, 128), _i32) for _ in range(8)]
    rows = []
    for c in range(_NCH):
        raccs = [jnp.zeros((_CH, 128), _i32) for _ in range(4)]
        for s in range(_NV):
            k = kvreg(c, s)
            le = _lt(k, kstar + 1)
            gaccs[s & 7] = gaccs[s & 7] + le
            raccs[s & 3] = raccs[s & 3] + (le - _lt(k, kstar))
        racc = (raccs[0] + raccs[1]) + (raccs[2] + raccs[3])
        rows.append(jnp.sum(racc, axis=1, keepdims=True))
    ga = gaccs
    while len(ga) > 1:
        ga = [ga[j] + ga[j + 1] for j in range(0, len(ga), 2)]
    count_gt = jnp.int32(_TOTAL) - jnp.sum(ga[0])
    rc = jnp.concatenate(rows, axis=0)
    r = kk - count_gt

    row_iota = jax.lax.broadcasted_iota(_i32, (_ROWS, 1), 0)

    def row_prefix(a):
        return jnp.sum(jnp.where(row_iota < a, rc, 0))

    def row_round(i, lo):
        cand = lo + (jnp.int32(64) >> i)
        return jnp.where(row_prefix(cand) < r, cand, lo)

    brow = jax.lax.fori_loop(0, 7, row_round, jnp.int32(0))
    rem = r - row_prefix(brow)

    eq_row = (
        jax.lax.bitcast_convert_type(o_ref[pl.ds(brow, 1), :], _i32) == kstar
    ).astype(_i32)
    col_iota = jax.lax.broadcasted_iota(_i32, (1, _COLS), 1)

    def col_prefix(c):
        return jnp.sum(jnp.where(col_iota < c, eq_row, 0))

    def col_round(i, lo):
        cand = lo + (jnp.int32(16384) >> i)
        return jnp.where(col_prefix(cand) < rem, cand, lo)

    locol = jax.lax.fori_loop(0, 15, col_round, jnp.int32(0))

    cut = jnp.where(
        row_iota < brow,
        jnp.int32(_COLS),
        jnp.where(row_iota == brow, locol + 1, jnp.int32(0)),
    )

    for c in range(_NCH):
        k = kchunk(c)
        cid = jax.lax.broadcasted_iota(_i32, (_CH, _COLS), 1)
        cutc = cut[c * _CH:(c + 1) * _CH, :]
        keep = (k > kstar) | ((k == kstar) & (cid < cutc))
        o_ref[c * _CH:(c + 1) * _CH, :] = jnp.where(
            keep, jax.lax.bitcast_convert_type(k, _f32), 0.0
        )


def kernel(x):
    hists = _sc_hist(x)
    return pl.pallas_call(
        _body,
        out_shape=jax.ShapeDtypeStruct((_ROWS, _COLS), jnp.float32),
        in_specs=[
            pl.BlockSpec((_ROWS, _COLS), lambda: (0, 0)),
            pl.BlockSpec((_NTILES, _BINS), lambda: (0, 0)),
        ],
        out_specs=pl.BlockSpec((_ROWS, _COLS), lambda: (0, 0)),
        scratch_shapes=[pltpu.VMEM((_CH, _BINS), jnp.int32)],
    )(x, hists)


# speedup vs baseline: 2.1903x; 2.1903x over previous
import jax
import jax.numpy as jnp
from jax.experimental import pallas as pl
from jax.experimental.pallas import tpu as pltpu

_ROWS = 128
_COLS = 32768
_TOTAL = _ROWS * _COLS
_KK = 64 * _ROWS
_CH = 8
_NCH = _ROWS // _CH
_SUB = 4096
_NSUB = _COLS // _SUB

_i32 = jnp.int32
_f32 = jnp.float32


def _lt(k, t):
    return jax.lax.shift_right_logical(k - t, 31)


def _body(x_ref, o_ref, k16_ref):
    for c in range(_ROWS // 16):
        xb = x_ref[c * 16:(c + 1) * 16, :]
        keys = jnp.maximum(jax.lax.bitcast_convert_type(xb, _i32), 0)
        o_ref[c * 16:(c + 1) * 16, :] = jax.lax.bitcast_convert_type(
            keys, _f32
        )
        k16_ref[c * 16:(c + 1) * 16, :] = jax.lax.shift_right_logical(
            keys, 16
        ).astype(jnp.int16)

    def kvreg(c, s):
        return jax.lax.bitcast_convert_type(
            o_ref[c * _CH:(c + 1) * _CH, s * 128:(s + 1) * 128], _i32
        )

    def kchunk(c):
        return jax.lax.bitcast_convert_type(
            o_ref[c * _CH:(c + 1) * _CH, :], _i32
        )

    _NV = _COLS // 128

    def count_lt(t):
        accs = [jnp.zeros((_CH, 128), _i32) for _ in range(8)]
        i = 0
        for c in range(_NCH):
            for s in range(_NV):
                accs[i & 7] = accs[i & 7] + _lt(kvreg(c, s), t)
                i += 1
        a = accs
        while len(a) > 1:
            a = [a[j] + a[j + 1] for j in range(0, len(a), 2)]
        return jnp.sum(a[0])

    def count_lt16(b):
        t16 = b.astype(jnp.int16)
        accs = [jnp.zeros((16, 128), jnp.int16) for _ in range(8)]
        i = 0
        for c in range(_ROWS // 16):
            for s in range(_NV):
                k16 = k16_ref[c * 16:(c + 1) * 16, s * 128:(s + 1) * 128]
                accs[i & 7] = accs[i & 7] + jnp.where(
                    k16 < t16, jnp.int16(1), jnp.int16(0)
                )
                i += 1
        a = accs
        while len(a) > 1:
            a = [a[j] + a[j + 1] for j in range(0, len(a), 2)]
        return jnp.sum(a[0].astype(_i32))

    kk = jnp.int32(_KK)
    ge_kk = jnp.int32(_TOTAL - _KK)

    def bin_round(i, cur):
        cand = cur + (jnp.int32(1) << (jnp.int32(14) - i))
        return jnp.where(count_lt16(cand) <= ge_kk, cand, cur)

    bstar = jax.lax.fori_loop(0, 15, bin_round, jnp.int32(0))

    def key_round(i, cur):
        cand = cur + (jnp.int32(1) << (jnp.int32(30) - i))
        return jnp.where(count_lt(cand) <= ge_kk, cand, cur)

    kstar = jax.lax.fori_loop(
        15, 31, key_round, jax.lax.shift_left(bstar, 16)
    )

    gaccs = [jnp.zeros((_CH, 128), _i32) for _ in range(8)]
    rows = []
    for c in range(_NCH):
        raccs = [jnp.zeros((_CH, 128), _i32) for _ in range(4)]
        for s in range(_NV):
            k = kvreg(c, s)
            le = _lt(k, kstar + 1)
            gaccs[s & 7] = gaccs[s & 7] + le
            raccs[s & 3] = raccs[s & 3] + (le - _lt(k, kstar))
        racc = (raccs[0] + raccs[1]) + (raccs[2] + raccs[3])
        rows.append(jnp.sum(racc, axis=1, keepdims=True))
    ga = gaccs
    while len(ga) > 1:
        ga = [ga[j] + ga[j + 1] for j in range(0, len(ga), 2)]
    count_gt = jnp.int32(_TOTAL) - jnp.sum(ga[0])
    rc = jnp.concatenate(rows, axis=0)
    r = kk - count_gt

    row_iota = jax.lax.broadcasted_iota(_i32, (_ROWS, 1), 0)

    def row_prefix(a):
        return jnp.sum(jnp.where(row_iota < a, rc, 0))

    def row_round(i, lo):
        cand = lo + (jnp.int32(64) >> i)
        return jnp.where(row_prefix(cand) < r, cand, lo)

    brow = jax.lax.fori_loop(0, 7, row_round, jnp.int32(0))
    rem = r - row_prefix(brow)

    eq_row = (
        jax.lax.bitcast_convert_type(o_ref[pl.ds(brow, 1), :], _i32) == kstar
    ).astype(_i32)
    col_iota = jax.lax.broadcasted_iota(_i32, (1, _COLS), 1)

    def col_prefix(c):
        return jnp.sum(jnp.where(col_iota < c, eq_row, 0))

    def col_round(i, lo):
        cand = lo + (jnp.int32(16384) >> i)
        return jnp.where(col_prefix(cand) < rem, cand, lo)

    locol = jax.lax.fori_loop(0, 15, col_round, jnp.int32(0))

    cut = jnp.where(
        row_iota < brow,
        jnp.int32(_COLS),
        jnp.where(row_iota == brow, locol + 1, jnp.int32(0)),
    )

    for c in range(_NCH):
        k = kchunk(c)
        cid = jax.lax.broadcasted_iota(_i32, (_CH, _COLS), 1)
        cutc = cut[c * _CH:(c + 1) * _CH, :]
        keep = (k > kstar) | ((k == kstar) & (cid < cutc))
        o_ref[c * _CH:(c + 1) * _CH, :] = jnp.where(
            keep, jax.lax.bitcast_convert_type(k, _f32), 0.0
        )


def kernel(x):
    return pl.pallas_call(
        _body,
        out_shape=jax.ShapeDtypeStruct((_ROWS, _COLS), jnp.float32),
        in_specs=[pl.BlockSpec((_ROWS, _COLS), lambda: (0, 0))],
        out_specs=pl.BlockSpec((_ROWS, _COLS), lambda: (0, 0)),
        scratch_shapes=[pltpu.VMEM((_ROWS, _COLS), jnp.int16)],
    )(x)


# speedup vs baseline: 2.2826x; 1.0421x over previous
import jax
import jax.numpy as jnp
from jax.experimental import pallas as pl
from jax.experimental.pallas import tpu as pltpu

_ROWS = 128
_COLS = 32768
_TOTAL = _ROWS * _COLS
_KK = 64 * _ROWS
_CH = 8
_NCH = _ROWS // _CH
_SUB = 4096
_NSUB = _COLS // _SUB

_i32 = jnp.int32
_f32 = jnp.float32


def _lt(k, t):
    return jax.lax.shift_right_logical(k - t, 31)


def _tree(m):
    w = m.shape[1]
    while w > 128:
        w //= 2
        m = m[:, :w] + m[:, w:]
    return m


def _body(x_ref, o_ref):
    for c in range(_NCH):
        xb = x_ref[c * _CH:(c + 1) * _CH, :]
        keys = jnp.maximum(jax.lax.bitcast_convert_type(xb, _i32), 0)
        o_ref[c * _CH:(c + 1) * _CH, :] = jax.lax.bitcast_convert_type(
            keys, _f32
        )

    def kchunk(c, s=None):
        if s is None:
            sl = slice(None)
        else:
            sl = slice(s * _SUB, (s + 1) * _SUB)
        return jax.lax.bitcast_convert_type(
            o_ref[c * _CH:(c + 1) * _CH, sl], _i32
        )

    def count_lt(t):
        vacc = jnp.zeros((_CH, 128), _i32)
        for c in range(_NCH):
            for s in range(_NSUB):
                vacc = vacc + _tree(_lt(kchunk(c, s), t))
        return jnp.sum(vacc)

    kk = jnp.int32(_KK)
    ge_kk = jnp.int32(_TOTAL - _KK)

    def key_round(i, cur):
        cand = cur + (jnp.int32(1) << (jnp.int32(30) - i))
        return jnp.where(count_lt(cand) <= ge_kk, cand, cur)

    kstar = jax.lax.fori_loop(0, 31, key_round, jnp.int32(0))

    vacc = jnp.zeros((_CH, 128), _i32)
    rows = []
    for c in range(_NCH):
        racc = jnp.zeros((_CH, 128), _i32)
        for s in range(_NSUB):
            k = kchunk(c, s)
            le = _lt(k, kstar + 1)
            vacc = vacc + _tree(le)
            racc = racc + _tree(le - _lt(k, kstar))
        rows.append(jnp.sum(racc, axis=1, keepdims=True))
    count_gt = jnp.int32(_TOTAL) - jnp.sum(vacc)
    rc = jnp.concatenate(rows, axis=0)
    r = kk - count_gt

    row_iota = jax.lax.broadcasted_iota(_i32, (_ROWS, 1), 0)

    def row_prefix(a):
        return jnp.sum(jnp.where(row_iota < a, rc, 0))

    def row_round(i, lo):
        cand = lo + (jnp.int32(64) >> i)
        return jnp.where(row_prefix(cand) < r, cand, lo)

    brow = jax.lax.fori_loop(0, 7, row_round, jnp.int32(0))
    rem = r - row_prefix(brow)

    eq_row = (
        jax.lax.bitcast_convert_type(o_ref[pl.ds(brow, 1), :], _i32) == kstar
    ).astype(_i32)
    col_iota = jax.lax.broadcasted_iota(_i32, (1, _COLS), 1)

    def col_prefix(c):
        return jnp.sum(jnp.where(col_iota < c, eq_row, 0))

    def col_round(i, lo):
        cand = lo + (jnp.int32(16384) >> i)
        return jnp.where(col_prefix(cand) < rem, cand, lo)

    locol = jax.lax.fori_loop(0, 15, col_round, jnp.int32(0))

    cut = jnp.where(
        row_iota < brow,
        jnp.int32(_COLS),
        jnp.where(row_iota == brow, locol + 1, jnp.int32(0)),
    )

    for c in range(_NCH):
        k = kchunk(c)
        cid = jax.lax.broadcasted_iota(_i32, (_CH, _COLS), 1)
        cutc = cut[c * _CH:(c + 1) * _CH, :]
        keep = (k > kstar) | ((k == kstar) & (cid < cutc))
        o_ref[c * _CH:(c + 1) * _CH, :] = jnp.where(
            keep, jax.lax.bitcast_convert_type(k, _f32), 0.0
        )


def kernel(x):
    return pl.pallas_call(
        _body,
        out_shape=jax.ShapeDtypeStruct((_ROWS, _COLS), jnp.float32),
        in_specs=[pl.BlockSpec((_ROWS, _COLS), lambda: (0, 0))],
        out_specs=pl.BlockSpec((_ROWS, _COLS), lambda: (0, 0)),
    )(x)
